# Initial kernel scaffold; baseline (speedup 1.0000x reference)
import jax
import jax.numpy as jnp
from jax.experimental import pallas as pl


def kernel(seq_len, relative_bias):
    raise NotImplementedError("write your pallas kernel here")


# speedup vs baseline: 40.2099x; 40.2099x over previous
import functools

import jax
import jax.numpy as jnp
from jax import lax
from jax.experimental import pallas as pl
from jax.experimental.pallas import tpu as pltpu
from jax.experimental.pallas import tpu_sc as plsc

N_HEADS = 16
MAX_DIST = 128
NREL = 2 * MAX_DIST + 1
SEQ = 2048
SHIFTS = 16
GW = 4096
EW = GW + SHIFTS
ROWS_PER_WORKER = (N_HEADS * SEQ) // 32
BLOCKS_PER_WORKER = ROWS_PER_WORKER // SHIFTS
FIRE = 8


def _build_g16(relative_bias):
    t = relative_bias
    left = jnp.broadcast_to(t[:, :1], (N_HEADS, SEQ - 1 - MAX_DIST))
    right_len = EW - (SEQ - 1 - MAX_DIST) - NREL
    right = jnp.broadcast_to(t[:, -1:], (N_HEADS, right_len))
    e = jnp.concatenate([left, t, right], axis=1)
    g16 = jnp.stack(
        [e[:, SHIFTS - 1 - d : SHIFTS - 1 - d + GW] for d in range(SHIFTS)],
        axis=1,
    )
    return g16


def _sc_body(g16_hbm, out_hbm, g16_vm, sem):
    c = lax.axis_index("c")
    s = lax.axis_index("s")
    wid = s * 2 + c
    h = wid // 2
    r0 = (wid % 2) * ROWS_PER_WORKER
    pltpu.sync_copy(g16_hbm.at[h], g16_vm)

    def body(t_, carry):
        i0 = r0 + t_ * (SHIFTS * FIRE)
        copies = []
        for u in range(FIRE):
            ii = i0 + u * SHIFTS
            off = (SEQ - SHIFTS) - ii
            cp = pltpu.make_async_copy(
                g16_vm.at[:, pl.ds(off, SEQ)],
                out_hbm.at[h, pl.ds(ii, SHIFTS)],
                sem,
            )
            cp.start()
            copies.append(cp)
        for cp in copies:
            cp.wait()
        return carry

    lax.fori_loop(0, BLOCKS_PER_WORKER // FIRE, body, 0)


def kernel(seq_len, relative_bias):
    del seq_len
    g16 = _build_g16(relative_bias)
    mesh = plsc.VectorSubcoreMesh(core_axis_name="c", subcore_axis_name="s")
    run = functools.partial(
        pl.kernel,
        mesh=mesh,
        out_type=jax.ShapeDtypeStruct((N_HEADS, SEQ, SEQ), jnp.float32),
        scratch_types=[
            pltpu.VMEM((SHIFTS, GW), jnp.float32),
            pltpu.SemaphoreType.DMA,
        ],
        compiler_params=pltpu.CompilerParams(use_tc_tiling_on_sc=False),
    )(_sc_body)
    return run(g16)
